# Initial kernel scaffold; baseline (speedup 1.0000x reference)
import jax
import jax.numpy as jnp
from jax.experimental import pallas as pl


def kernel(loc_data, conf_data, priors, targets):
    raise NotImplementedError("write your pallas kernel here")


# speedup vs baseline: 24.1152x; 24.1152x over previous
import jax
import jax.numpy as jnp
from jax.experimental import pallas as pl
from jax.experimental.pallas import tpu as pltpu

_B = 32
_P = 5415
_C = 21
_T = 8
_CELLS = 361
_A = 15
_V0 = 0.1
_V1 = 0.2


def _match_kernel(pri_ref, tgt_ref, loc_ref, ct_ref, np_ref, ll_ref):
    pcx = pri_ref[0:1, :]
    pcy = pri_ref[1:2, :]
    pw = pri_ref[2:3, :]
    ph = pri_ref[3:4, :]
    px1 = pcx - pw / 2.0
    py1 = pcy - ph / 2.0
    px2 = pcx + pw / 2.0
    py2 = pcy + ph / 2.0
    area_b = (px2 - px1) * (py2 - py1)
    lane = jax.lax.broadcasted_iota(jnp.int32, (_B, _P), 1).astype(jnp.float32)

    bto = jnp.full((_B, _P), -1.0, dtype=jnp.float32)
    bti = jnp.zeros((_B, _P), dtype=jnp.float32)
    best_prior = []
    for t in range(_T):
        tx1 = tgt_ref[:, 5 * t + 0:5 * t + 1]
        ty1 = tgt_ref[:, 5 * t + 1:5 * t + 2]
        tx2 = tgt_ref[:, 5 * t + 2:5 * t + 3]
        ty2 = tgt_ref[:, 5 * t + 3:5 * t + 4]
        ix1 = jnp.maximum(px1, tx1)
        iy1 = jnp.maximum(py1, ty1)
        ix2 = jnp.minimum(px2, tx2)
        iy2 = jnp.minimum(py2, ty2)
        iw = jnp.maximum(ix2 - ix1, 0.0)
        ih = jnp.maximum(iy2 - iy1, 0.0)
        inter = iw * ih
        area_a = (tx2 - tx1) * (ty2 - ty1)
        iou = inter / (area_a + area_b - inter)
        upd = iou > bto
        bti = jnp.where(upd, float(t), bti)
        bto = jnp.where(upd, iou, bto)
        m = jnp.max(iou, axis=1, keepdims=True)
        bp = jnp.min(jnp.where(iou == m, lane, 1e9), axis=1, keepdims=True)
        best_prior.append(bp)
    for t in range(_T):
        f = lane == best_prior[t]
        bto = jnp.where(f, 2.0, bto)
        bti = jnp.where(f, float(t), bti)

    lab = jnp.zeros((_B, _P), dtype=jnp.float32)
    mx1 = jnp.zeros((_B, _P), dtype=jnp.float32)
    my1 = jnp.zeros((_B, _P), dtype=jnp.float32)
    mx2 = jnp.zeros((_B, _P), dtype=jnp.float32)
    my2 = jnp.zeros((_B, _P), dtype=jnp.float32)
    for t in range(_T):
        sel = bti == float(t)
        lab = jnp.where(sel, tgt_ref[:, 5 * t + 4:5 * t + 5], lab)
        mx1 = jnp.where(sel, tgt_ref[:, 5 * t + 0:5 * t + 1], mx1)
        my1 = jnp.where(sel, tgt_ref[:, 5 * t + 1:5 * t + 2], my1)
        mx2 = jnp.where(sel, tgt_ref[:, 5 * t + 2:5 * t + 3], mx2)
        my2 = jnp.where(sel, tgt_ref[:, 5 * t + 3:5 * t + 4], my2)
    conf = jnp.where(bto < 0.5, 0.0, lab + 1.0)
    ct_ref[:, :] = conf
    pos = bto >= 0.5
    np_ref[:, :] = jnp.sum(jnp.where(pos, 1.0, 0.0), axis=1, keepdims=True)

    g_cx = ((mx1 + mx2) / 2.0 - pcx) / (_V0 * pw)
    g_cy = ((my1 + my2) / 2.0 - pcy) / (_V0 * ph)
    g_w = jnp.log((mx2 - mx1) / pw) / _V1
    g_h = jnp.log((my2 - my1) / ph) / _V1
    total = jnp.float32(0.0)
    for j, g in enumerate((g_cx, g_cy, g_w, g_h)):
        d = loc_ref[j] - g
        ad = jnp.abs(d)
        sl1 = jnp.where(ad < 1.0, 0.5 * ad * ad, ad - 0.5)
        total = total + jnp.sum(jnp.where(pos, sl1, 0.0))
    ll_ref[:, :] = total.reshape(1, 1)


def _conf_kernel(conf_ref, ct_ref, lcm_ref, ps_ref):
    b = pl.program_id(0)
    x = conf_ref[0]
    m = jnp.max(x, axis=0, keepdims=True)
    s = jnp.sum(jnp.exp(x - m), axis=0, keepdims=True)
    lse = m + jnp.log(s)
    ct = ct_ref[0]
    cls = jax.lax.broadcasted_iota(jnp.int32, (_C, _P), 0).astype(jnp.float32)
    gathered = jnp.sum(jnp.where(cls == ct, x, 0.0), axis=0, keepdims=True)
    raw = lse - gathered
    pos = ct > 0.0

    @pl.when(b == 0)
    def _():
        ps_ref[:, :] = jnp.zeros((1, 1), dtype=jnp.float32)

    ps_ref[:, :] += jnp.sum(jnp.where(pos, raw, 0.0)).reshape(1, 1)
    lcm_ref[...] = jnp.where(pos, 0.0, jnp.maximum(raw, 0.0)).reshape(1, 1, _P)


def _mine_kernel(lcm_ref, np_ref, lcm3_ref, ct3_ref, fm_ref, hv_ref, ns_ref):
    bits = jax.lax.bitcast_convert_type(lcm_ref[...], jnp.int32)
    npos = np_ref[...]
    k = jnp.minimum(3.0 * npos, float(_P - 1))
    v = jnp.zeros((_B, 1), dtype=jnp.int32)
    for bit in range(30, -1, -1):
        cand = v | (1 << bit)
        cnt = jnp.sum(jnp.where(bits >= cand, 1.0, 0.0), axis=1, keepdims=True)
        v = jnp.where(cnt >= k, cand, v)
    need = k - jnp.sum(jnp.where(bits > v, 1.0, 0.0), axis=1, keepdims=True)
    tie = bits == v
    lane = jax.lax.broadcasted_iota(jnp.int32, (_B, _P), 1)
    idx = jnp.zeros((_B, 1), dtype=jnp.int32)
    for bit in range(12, -1, -1):
        cand = idx | (1 << bit)
        cnt = jnp.sum(jnp.where(tie & (lane < cand), 1.0, 0.0),
                      axis=1, keepdims=True)
        idx = jnp.where(cnt <= need, cand, idx)

    lcm3 = lcm3_ref[...]
    bits3 = jax.lax.bitcast_convert_type(lcm3, jnp.int32)
    ct3 = ct3_ref[...]
    v3 = v.reshape(_B, 1, 1)
    i3 = idx.reshape(_B, 1, 1)
    cell = jax.lax.broadcasted_iota(jnp.int32, (_B, _CELLS, _A), 1)
    anc = jax.lax.broadcasted_iota(jnp.int32, (_B, _CELLS, _A), 2)
    p3 = cell * _A + anc
    neg3 = (bits3 > v3) | ((bits3 == v3) & (p3 < i3))
    pos3 = ct3 > 0.0
    mask3 = pos3 | neg3
    ns_ref[:, :] = jnp.sum(jnp.where(neg3, lcm3, 0.0)).reshape(1, 1)
    fm_ref[...] = jnp.max(ct3, axis=2)
    hv_ref[...] = jnp.max(jnp.where(mask3, 1.0, 0.0), axis=2)


def kernel(loc_data, conf_data, priors, targets):
    pri_t = priors.T
    tgt = targets.reshape(_B, _T * 5)
    loc_t3 = jnp.transpose(loc_data, (2, 0, 1))
    conf_t3 = jnp.transpose(conf_data, (0, 2, 1))

    ct, npos, ll = pl.pallas_call(
        _match_kernel,
        out_shape=[
            jax.ShapeDtypeStruct((_B, _P), jnp.float32),
            jax.ShapeDtypeStruct((_B, 1), jnp.float32),
            jax.ShapeDtypeStruct((1, 1), jnp.float32),
        ],
    )(pri_t, tgt, loc_t3)

    lcm, psum = pl.pallas_call(
        _conf_kernel,
        grid=(_B,),
        in_specs=[
            pl.BlockSpec((1, _C, _P), lambda b: (b, 0, 0)),
            pl.BlockSpec((1, 1, _P), lambda b: (b, 0, 0)),
        ],
        out_specs=[
            pl.BlockSpec((1, 1, _P), lambda b: (b, 0, 0)),
            pl.BlockSpec((1, 1), lambda b: (0, 0)),
        ],
        out_shape=[
            jax.ShapeDtypeStruct((_B, 1, _P), jnp.float32),
            jax.ShapeDtypeStruct((1, 1), jnp.float32),
        ],
    )(conf_t3, ct.reshape(_B, 1, _P))
    lcm = lcm.reshape(_B, _P)

    fm, hv, nsum = pl.pallas_call(
        _mine_kernel,
        out_shape=[
            jax.ShapeDtypeStruct((_B, _CELLS), jnp.float32),
            jax.ShapeDtypeStruct((_B, _CELLS), jnp.float32),
            jax.ShapeDtypeStruct((1, 1), jnp.float32),
        ],
    )(lcm, npos, lcm.reshape(_B, _CELLS, _A), ct.reshape(_B, _CELLS, _A))

    n = jnp.sum(npos)
    loss_l = ll[0, 0] / n
    loss_c = (psum[0, 0] + nsum[0, 0]) / n
    conf_t_featuremap = fm.reshape(-1).astype(jnp.int32)
    have_centerloss = hv.reshape(-1) > 0.5
    return (loss_l, loss_c, conf_t_featuremap, have_centerloss)
